# baseline (speedup 1.0000x reference)
import jax
import jax.numpy as jnp
from jax.experimental import pallas as pl

_BS = 512


def _pe_add_kernel(x_ref, pe_ref, out_ref):
    x = x_ref[...]
    pe = pe_ref[...]
    out_ref[...] = jnp.where(x == 0.0, x, x + pe[None, :, :])


def kernel(x, pos_embed):
    batch, seq, dim = x.shape
    pe = pos_embed[:seq]
    grid = (seq // _BS, batch)
    return pl.pallas_call(
        _pe_add_kernel,
        grid=grid,
        in_specs=[
            pl.BlockSpec((1, _BS, dim), lambda s, b: (b, s, 0)),
            pl.BlockSpec((_BS, dim), lambda s, b: (s, 0)),
        ],
        out_specs=pl.BlockSpec((1, _BS, dim), lambda s, b: (b, s, 0)),
        out_shape=jax.ShapeDtypeStruct(x.shape, x.dtype),
    )(x, pe)


# speedup vs baseline: 1.2344x; 1.2344x over previous
import jax
import jax.numpy as jnp
from jax.experimental import pallas as pl

_BS = 2048


def _pe_add_kernel(x_ref, pe_ref, out_ref):
    x = x_ref[...]
    pe = pe_ref[...]
    out_ref[...] = jnp.where(x == 0.0, x, x + pe[None, :, :])


def kernel(x, pos_embed):
    batch, seq, dim = x.shape
    pe = pos_embed[:seq]
    grid = (seq // _BS, batch)
    return pl.pallas_call(
        _pe_add_kernel,
        grid=grid,
        in_specs=[
            pl.BlockSpec((1, _BS, dim), lambda s, b: (b, s, 0)),
            pl.BlockSpec((_BS, dim), lambda s, b: (s, 0)),
        ],
        out_specs=pl.BlockSpec((1, _BS, dim), lambda s, b: (b, s, 0)),
        out_shape=jax.ShapeDtypeStruct(x.shape, x.dtype),
    )(x, pe)
